# baseline (speedup 1.0000x reference)
import functools

import jax
import jax.numpy as jnp
from jax import lax
from jax.experimental import pallas as pl
from jax.experimental.pallas import tpu as pltpu
from jax.experimental.pallas import tpu_sc as plsc

N = 10000
NPAD = 10240
D = 128
E = 160000
BLK = 2048
GRID = NPAD // BLK

NTILES = 16
ROWS = 80
COLS = 128
EPAD = NTILES * ROWS * COLS
SL = NPAD // NTILES
NH = NPAD // 2
TRASH = 128
SLH = NH // NTILES


def _splat_i32(v):
    return jnp.full((16,), v, jnp.int32)


def _rsqrt16(d):
    y = jnp.ones((16,), jnp.float32)
    for _ in range(10):
        y = jnp.where(d * y * y > 1.9, y * 0.5, y)
    for _ in range(5):
        y = y * (1.5 - 0.5 * d * y * y)
    return y


def _sc_body(x_hbm, edges_hbm, w_hbm, zeros_hbm, p_out, t_out,
             src2d, dst2d, wv, sv, abuf, bbuf, xb, xb2, idxb, tbuf, robuf,
             deg_o, deg_i, s_acc, p_acc, sem0, sem1):
    c = lax.axis_index("c")
    s = lax.axis_index("s")
    nbase = s * SL

    zero16 = jnp.zeros((16,), jnp.float32)

    def _z(i, carry):
        tbuf[pl.ds(i * 16, 16)] = zero16
        return carry

    lax.fori_loop(0, SL // 16, _z, 0)
    pltpu.sync_copy(tbuf, deg_o.at[pl.ds(nbase, SL)])
    pltpu.sync_copy(tbuf, deg_i.at[pl.ds(nbase, SL)])
    pltpu.sync_copy(tbuf, s_acc.at[pl.ds(nbase, SL)])
    pltpu.sync_copy(zeros_hbm, p_acc.at[pl.ds(s * SLH, SLH)])
    pltpu.sync_copy(edges_hbm.at[c * 2 * NTILES + s], src2d)
    pltpu.sync_copy(edges_hbm.at[c * 2 * NTILES + NTILES + s], dst2d)
    pltpu.sync_copy(w_hbm.at[c * NTILES + s], wv)
    ones16 = jnp.ones((16,), jnp.float32)
    for q in range(COLS // 16):
        sv[pl.ds(q * 16, 16)] = ones16
    plsc.subcore_barrier()

    def _deg(j, carry):
        pltpu.sync_copy(sv, deg_o.at[src2d.at[j]], add=True)
        pltpu.sync_copy(sv, deg_i.at[dst2d.at[j]], add=True)
        return carry

    lax.fori_loop(0, ROWS, _deg, 0)
    plsc.subcore_barrier()

    for ref in (deg_o, deg_i):
        pltpu.sync_copy(ref.at[pl.ds(nbase, SL)], tbuf)

        def _rs(i, carry):
            d = jnp.maximum(tbuf[pl.ds(i * 16, 16)], 1.0)
            tbuf[pl.ds(i * 16, 16)] = _rsqrt16(d)
            return carry

        lax.fori_loop(0, SL // 16, _rs, 0)
        pltpu.sync_copy(tbuf, ref.at[pl.ds(nbase, SL)])
    plsc.subcore_barrier()

    def _srow(j, carry):
        pltpu.sync_copy(deg_o.at[src2d.at[j]], abuf)
        pltpu.sync_copy(deg_i.at[dst2d.at[j]], bbuf)
        for q in range(COLS // 16):
            ds_ = pl.ds(q * 16, 16)
            w16 = wv[j, ds_]
            b16 = bbuf[ds_]
            sv[ds_] = w16 * b16
            wv[j, ds_] = w16 * abuf[ds_] * b16
        pltpu.sync_copy(sv, s_acc.at[src2d.at[j]], add=True)
        return carry

    lax.fori_loop(0, ROWS, _srow, 0)

    iota16 = lax.iota(jnp.int32, 16)
    nb2 = s * SLH
    for h in range(2):
        base = jnp.int32(h * NH)

        def _do_chunk(j, buf, sem):
            pltpu.make_async_copy(x_hbm.at[src2d.at[j]], buf, sem).wait()
            for q in range(COLS // 16):
                ds_ = pl.ds(q * 16, 16)
                d16 = dst2d[j, ds_] - base
                trash = NH + q * 16 + iota16
                inr = (d16 >= 0) & (d16 < NH)
                idxb[0, ds_] = jnp.where(inr, d16, trash)

            def _sgroup(q, qc):
                ds_ = pl.ds(q * 16, 16)
                w16 = wv[j, ds_]
                d16 = dst2d[j, ds_] - base
                inr = jnp.where((d16 >= 0) & (d16 < NH), 1, 0)
                for i in range(16):
                    k = q * 16 + i

                    @pl.when(inr[i] == 1)
                    def _():
                        wsp = jnp.broadcast_to(w16[i], (16,))
                        for g in range(D // 16):
                            dsg = pl.ds(g * 16, 16)
                            buf[k, dsg] = buf[k, dsg] * wsp
                return qc

            lax.fori_loop(0, COLS // 16, _sgroup, 0)
            pltpu.sync_copy(buf, p_acc.at[idxb.at[0]], add=True)

        pltpu.async_copy(x_hbm.at[src2d.at[0]], xb, sem0)

        def _vrow2(j2, carry):
            j = 2 * j2
            pltpu.async_copy(x_hbm.at[src2d.at[j + 1]], xb2, sem1)
            _do_chunk(j, xb, sem0)

            @pl.when(j2 < ROWS // 2 - 1)
            def _():
                pltpu.async_copy(x_hbm.at[src2d.at[j + 2]], xb, sem0)

            _do_chunk(j + 1, xb2, sem1)
            return carry

        lax.fori_loop(0, ROWS // 2, _vrow2, 0)
        plsc.subcore_barrier()
        pltpu.sync_copy(p_acc.at[pl.ds(nb2, SLH)],
                        p_out.at[c, h, pl.ds(nb2, SLH)])
        if h == 0:
            pltpu.sync_copy(zeros_hbm, p_acc.at[pl.ds(nb2, SLH)])
            plsc.subcore_barrier()

    pltpu.sync_copy(s_acc.at[pl.ds(nbase, SL)], tbuf)
    pltpu.sync_copy(deg_o.at[pl.ds(nbase, SL)], robuf)

    def _tm(i, carry):
        ds_ = pl.ds(i * 16, 16)
        tbuf[ds_] = tbuf[ds_] * robuf[ds_]
        return carry

    lax.fori_loop(0, SL // 16, _tm, 0)
    pltpu.sync_copy(tbuf, t_out.at[c, pl.ds(nbase, SL)])


@functools.partial(
    pl.kernel,
    out_type=(jax.ShapeDtypeStruct((2, 2, NH, D), jnp.float32),
              jax.ShapeDtypeStruct((2, NPAD), jnp.float32)),
    mesh=plsc.VectorSubcoreMesh(core_axis_name="c", subcore_axis_name="s"),
    compiler_params=pltpu.CompilerParams(needs_layout_passes=False),
    scratch_types=(
        pltpu.VMEM((ROWS, COLS), jnp.int32),
        pltpu.VMEM((ROWS, COLS), jnp.int32),
        pltpu.VMEM((ROWS, COLS), jnp.float32),
        pltpu.VMEM((COLS,), jnp.float32),
        pltpu.VMEM((COLS,), jnp.float32),
        pltpu.VMEM((COLS,), jnp.float32),
        pltpu.VMEM((COLS, D), jnp.float32),
        pltpu.VMEM((COLS, D), jnp.float32),
        pltpu.VMEM((1, COLS), jnp.int32),
        pltpu.VMEM((SL,), jnp.float32),
        pltpu.VMEM((SL,), jnp.float32),
        pltpu.VMEM_SHARED((NPAD,), jnp.float32),
        pltpu.VMEM_SHARED((NPAD,), jnp.float32),
        pltpu.VMEM_SHARED((NPAD,), jnp.float32),
        pltpu.VMEM_SHARED((NH + TRASH, D), jnp.float32),
        pltpu.SemaphoreType.DMA,
        pltpu.SemaphoreType.DMA,
    ),
)
def _sc_prop(x_hbm, edges_hbm, w_hbm, zeros_hbm, p_out, t_out, *rest):
    _sc_body(x_hbm, edges_hbm, w_hbm, zeros_hbm, p_out, t_out, *rest)


def _readout_body(p_ref, t_ref, w0_ref, w1_ref, bsum_ref, out_ref, u_ref):
    i = pl.program_id(0)

    @pl.when(i == 0)
    def _():
        u_ref[...] = jnp.zeros_like(u_ref)

    h = jnp.dot(p_ref[0], w0_ref[0], preferred_element_type=jnp.float32)
    h = h + jnp.dot(p_ref[1], w0_ref[1], preferred_element_type=jnp.float32)
    h = jnp.maximum(h + bsum_ref[0][None, :], 0.0)
    u_ref[...] += jnp.dot(t_ref[...], h, preferred_element_type=jnp.float32)

    @pl.when(i == GRID - 1)
    def _():
        u = u_ref[...]
        out = jnp.dot(u[0:1], w1_ref[0], preferred_element_type=jnp.float32)
        out = out + jnp.dot(u[1:2], w1_ref[1], preferred_element_type=jnp.float32)
        out_ref[...] = out * (1.0 / N) + bsum_ref[1][None, :]


def _readout(p, t, w0, w1, bsum, interpret=False):
    return pl.pallas_call(
        _readout_body,
        grid=(GRID,),
        in_specs=[
            pl.BlockSpec((2, BLK, D), lambda i: (0, i, 0)),
            pl.BlockSpec((2, BLK), lambda i: (0, i)),
            pl.BlockSpec((2, D, D), lambda i: (0, 0, 0)),
            pl.BlockSpec((2, D, D), lambda i: (0, 0, 0)),
            pl.BlockSpec((2, D), lambda i: (0, 0)),
        ],
        out_specs=pl.BlockSpec((1, D), lambda i: (0, 0)),
        out_shape=jax.ShapeDtypeStruct((1, D), jnp.float32),
        scratch_shapes=[pltpu.VMEM((2, D), jnp.float32)],
        interpret=interpret,
    )(p, t, w0, w1, bsum)


def _propagate_sc(x, edges, weights):
    npadex = EPAD - E
    pad_nodes = (N + jnp.arange(npadex, dtype=jnp.int32) % (NPAD - N))
    pad_idx = jnp.broadcast_to(pad_nodes, (2, 2, npadex))
    edges_p = jnp.concatenate([edges, pad_idx], axis=2)
    weights_p = jnp.concatenate(
        [weights, jnp.zeros((2, npadex), jnp.float32)], axis=1)
    edges3 = edges_p.reshape(2 * 2 * NTILES, ROWS, COLS)
    w3 = weights_p.reshape(2 * NTILES, ROWS, COLS)
    zeros2d = jnp.zeros((SLH, D), jnp.float32)
    return _sc_prop(x, edges3, w3, zeros2d)


def _run(x, edges, weights, w0, w1, bsum, interpret=False):
    p, t = _propagate_sc(x, edges, weights)
    p = p.reshape(2, NPAD, D)
    return _readout(p, t, w0, w1, bsum, interpret=interpret)


def kernel(x, edge_index_follows, edge_weight_follows, edge_index_likes,
           edge_weight_likes, W0_follows, b0_follows, W0_likes, b0_likes,
           W1_follows, b1_follows, W1_likes, b1_likes):
    edges = jnp.stack([edge_index_follows, edge_index_likes])
    weights = jnp.stack([edge_weight_follows, edge_weight_likes])
    w0 = jnp.stack([W0_follows, W0_likes])
    w1 = jnp.stack([W1_follows, W1_likes])
    bsum = jnp.stack([b0_follows + b0_likes, b1_follows + b1_likes])
    return _run(x, edges, weights, w0, w1, bsum)


# speedup vs baseline: 1.2041x; 1.2041x over previous
import functools

import jax
import jax.numpy as jnp
from jax import lax
from jax.experimental import pallas as pl
from jax.experimental.pallas import tpu as pltpu
from jax.experimental.pallas import tpu_sc as plsc

N = 10000
NPAD = 10240
D = 128
E = 160000
BLK = 2048
GRID = NPAD // BLK

NTILES = 16
ROWS = 80
COLS = 128
EPAD = NTILES * ROWS * COLS
SL = NPAD // NTILES
NH = NPAD // 2
TRASH = 128
SLH = NH // NTILES


def _splat_i32(v):
    return jnp.full((16,), v, jnp.int32)


def _rsqrt16(d):
    y = jnp.ones((16,), jnp.float32)
    for _ in range(10):
        y = jnp.where(d * y * y > 1.9, y * 0.5, y)
    for _ in range(5):
        y = y * (1.5 - 0.5 * d * y * y)
    return y


def _sc_body(x_hbm, edges_hbm, w_hbm, zeros_hbm, p_out, t_out,
             src2d, dst2d, wv, sv, abuf, bbuf, xb, xb2, idxb, tbuf, robuf,
             deg_o, deg_i, s_acc, p_acc, sem0, sem1):
    c = lax.axis_index("c")
    s = lax.axis_index("s")
    nbase = s * SL

    zero16 = jnp.zeros((16,), jnp.float32)

    def _z(i, carry):
        tbuf[pl.ds(i * 16, 16)] = zero16
        return carry

    lax.fori_loop(0, SL // 16, _z, 0)
    pltpu.sync_copy(tbuf, deg_o.at[pl.ds(nbase, SL)])
    pltpu.sync_copy(tbuf, deg_i.at[pl.ds(nbase, SL)])
    pltpu.sync_copy(tbuf, s_acc.at[pl.ds(nbase, SL)])
    pltpu.sync_copy(zeros_hbm, p_acc.at[pl.ds(s * SLH, SLH)])
    pltpu.sync_copy(edges_hbm.at[c * 2 * NTILES + s], src2d)
    pltpu.sync_copy(edges_hbm.at[c * 2 * NTILES + NTILES + s], dst2d)
    pltpu.sync_copy(w_hbm.at[c * NTILES + s], wv)
    ones16 = jnp.ones((16,), jnp.float32)
    for q in range(COLS // 16):
        sv[pl.ds(q * 16, 16)] = ones16
    plsc.subcore_barrier()

    def _deg(j, carry):
        pltpu.sync_copy(sv, deg_o.at[src2d.at[j]], add=True)
        pltpu.sync_copy(sv, deg_i.at[dst2d.at[j]], add=True)
        return carry

    lax.fori_loop(0, ROWS, _deg, 0)
    plsc.subcore_barrier()

    for ref in (deg_o, deg_i):
        pltpu.sync_copy(ref.at[pl.ds(nbase, SL)], tbuf)

        def _rs(i, carry):
            d = jnp.maximum(tbuf[pl.ds(i * 16, 16)], 1.0)
            tbuf[pl.ds(i * 16, 16)] = _rsqrt16(d)
            return carry

        lax.fori_loop(0, SL // 16, _rs, 0)
        pltpu.sync_copy(tbuf, ref.at[pl.ds(nbase, SL)])
    plsc.subcore_barrier()

    def _srow(j, carry):
        pltpu.sync_copy(deg_o.at[src2d.at[j]], abuf)
        pltpu.sync_copy(deg_i.at[dst2d.at[j]], bbuf)
        for q in range(COLS // 16):
            ds_ = pl.ds(q * 16, 16)
            w16 = wv[j, ds_]
            b16 = bbuf[ds_]
            sv[ds_] = w16 * b16
            wv[j, ds_] = w16 * abuf[ds_] * b16
        pltpu.sync_copy(sv, s_acc.at[src2d.at[j]], add=True)
        return carry

    lax.fori_loop(0, ROWS, _srow, 0)

    iota16 = lax.iota(jnp.int32, 16)
    nb2 = s * SLH
    for h in range(2):
        base = jnp.int32(h * NH)

        def _do_chunk(j, buf, sem):
            pltpu.make_async_copy(x_hbm.at[src2d.at[j]], buf, sem).wait()
            for q in range(COLS // 16):
                ds_ = pl.ds(q * 16, 16)
                d16 = dst2d[j, ds_] - base
                trash = NH + q * 16 + iota16
                inr = (d16 >= 0) & (d16 < NH)
                idxb[0, ds_] = jnp.where(inr, d16, trash)

            def _sgroup(q, qc):
                w16 = wv[j, pl.ds(q * 16, 16)]
                for i in range(16):
                    k = q * 16 + i
                    wsp = jnp.broadcast_to(w16[i], (16,))
                    for g in range(D // 16):
                        dsg = pl.ds(g * 16, 16)
                        buf[k, dsg] = buf[k, dsg] * wsp
                return qc

            lax.fori_loop(0, COLS // 16, _sgroup, 0)
            pltpu.sync_copy(buf, p_acc.at[idxb.at[0]], add=True)

        pltpu.async_copy(x_hbm.at[src2d.at[0]], xb, sem0)

        def _vrow2(j2, carry):
            j = 2 * j2
            pltpu.async_copy(x_hbm.at[src2d.at[j + 1]], xb2, sem1)
            _do_chunk(j, xb, sem0)

            @pl.when(j2 < ROWS // 2 - 1)
            def _():
                pltpu.async_copy(x_hbm.at[src2d.at[j + 2]], xb, sem0)

            _do_chunk(j + 1, xb2, sem1)
            return carry

        lax.fori_loop(0, ROWS // 2, _vrow2, 0)
        plsc.subcore_barrier()
        pltpu.sync_copy(p_acc.at[pl.ds(nb2, SLH)],
                        p_out.at[c, h, pl.ds(nb2, SLH)])
        if h == 0:
            pltpu.sync_copy(zeros_hbm, p_acc.at[pl.ds(nb2, SLH)])
            plsc.subcore_barrier()

    pltpu.sync_copy(s_acc.at[pl.ds(nbase, SL)], tbuf)
    pltpu.sync_copy(deg_o.at[pl.ds(nbase, SL)], robuf)

    def _tm(i, carry):
        ds_ = pl.ds(i * 16, 16)
        tbuf[ds_] = tbuf[ds_] * robuf[ds_]
        return carry

    lax.fori_loop(0, SL // 16, _tm, 0)
    pltpu.sync_copy(tbuf, t_out.at[c, pl.ds(nbase, SL)])


@functools.partial(
    pl.kernel,
    out_type=(jax.ShapeDtypeStruct((2, 2, NH, D), jnp.float32),
              jax.ShapeDtypeStruct((2, NPAD), jnp.float32)),
    mesh=plsc.VectorSubcoreMesh(core_axis_name="c", subcore_axis_name="s"),
    compiler_params=pltpu.CompilerParams(needs_layout_passes=False),
    scratch_types=(
        pltpu.VMEM((ROWS, COLS), jnp.int32),
        pltpu.VMEM((ROWS, COLS), jnp.int32),
        pltpu.VMEM((ROWS, COLS), jnp.float32),
        pltpu.VMEM((COLS,), jnp.float32),
        pltpu.VMEM((COLS,), jnp.float32),
        pltpu.VMEM((COLS,), jnp.float32),
        pltpu.VMEM((COLS, D), jnp.float32),
        pltpu.VMEM((COLS, D), jnp.float32),
        pltpu.VMEM((1, COLS), jnp.int32),
        pltpu.VMEM((SL,), jnp.float32),
        pltpu.VMEM((SL,), jnp.float32),
        pltpu.VMEM_SHARED((NPAD,), jnp.float32),
        pltpu.VMEM_SHARED((NPAD,), jnp.float32),
        pltpu.VMEM_SHARED((NPAD,), jnp.float32),
        pltpu.VMEM_SHARED((NH + TRASH, D), jnp.float32),
        pltpu.SemaphoreType.DMA,
        pltpu.SemaphoreType.DMA,
    ),
)
def _sc_prop(x_hbm, edges_hbm, w_hbm, zeros_hbm, p_out, t_out, *rest):
    _sc_body(x_hbm, edges_hbm, w_hbm, zeros_hbm, p_out, t_out, *rest)


def _readout_body(p_ref, t_ref, w0_ref, w1_ref, bsum_ref, out_ref, u_ref):
    i = pl.program_id(0)

    @pl.when(i == 0)
    def _():
        u_ref[...] = jnp.zeros_like(u_ref)

    h = jnp.dot(p_ref[0], w0_ref[0], preferred_element_type=jnp.float32)
    h = h + jnp.dot(p_ref[1], w0_ref[1], preferred_element_type=jnp.float32)
    h = jnp.maximum(h + bsum_ref[0][None, :], 0.0)
    u_ref[...] += jnp.dot(t_ref[...], h, preferred_element_type=jnp.float32)

    @pl.when(i == GRID - 1)
    def _():
        u = u_ref[...]
        out = jnp.dot(u[0:1], w1_ref[0], preferred_element_type=jnp.float32)
        out = out + jnp.dot(u[1:2], w1_ref[1], preferred_element_type=jnp.float32)
        out_ref[...] = out * (1.0 / N) + bsum_ref[1][None, :]


def _readout(p, t, w0, w1, bsum, interpret=False):
    return pl.pallas_call(
        _readout_body,
        grid=(GRID,),
        in_specs=[
            pl.BlockSpec((2, BLK, D), lambda i: (0, i, 0)),
            pl.BlockSpec((2, BLK), lambda i: (0, i)),
            pl.BlockSpec((2, D, D), lambda i: (0, 0, 0)),
            pl.BlockSpec((2, D, D), lambda i: (0, 0, 0)),
            pl.BlockSpec((2, D), lambda i: (0, 0)),
        ],
        out_specs=pl.BlockSpec((1, D), lambda i: (0, 0)),
        out_shape=jax.ShapeDtypeStruct((1, D), jnp.float32),
        scratch_shapes=[pltpu.VMEM((2, D), jnp.float32)],
        interpret=interpret,
    )(p, t, w0, w1, bsum)


def _propagate_sc(x, edges, weights):
    npadex = EPAD - E
    pad_nodes = (N + jnp.arange(npadex, dtype=jnp.int32) % (NPAD - N))
    pad_idx = jnp.broadcast_to(pad_nodes, (2, 2, npadex))
    edges_p = jnp.concatenate([edges, pad_idx], axis=2)
    weights_p = jnp.concatenate(
        [weights, jnp.zeros((2, npadex), jnp.float32)], axis=1)
    edges3 = edges_p.reshape(2 * 2 * NTILES, ROWS, COLS)
    w3 = weights_p.reshape(2 * NTILES, ROWS, COLS)
    zeros2d = jnp.zeros((SLH, D), jnp.float32)
    return _sc_prop(x, edges3, w3, zeros2d)


def _run(x, edges, weights, w0, w1, bsum, interpret=False):
    p, t = _propagate_sc(x, edges, weights)
    p = p.reshape(2, NPAD, D)
    return _readout(p, t, w0, w1, bsum, interpret=interpret)


def kernel(x, edge_index_follows, edge_weight_follows, edge_index_likes,
           edge_weight_likes, W0_follows, b0_follows, W0_likes, b0_likes,
           W1_follows, b1_follows, W1_likes, b1_likes):
    edges = jnp.stack([edge_index_follows, edge_index_likes])
    weights = jnp.stack([edge_weight_follows, edge_weight_likes])
    w0 = jnp.stack([W0_follows, W0_likes])
    w1 = jnp.stack([W1_follows, W1_likes])
    bsum = jnp.stack([b0_follows + b0_likes, b1_follows + b1_likes])
    return _run(x, edges, weights, w0, w1, bsum)
